# baseline (speedup 1.0000x reference)
import functools

import jax
import jax.numpy as jnp
from jax import lax
from jax.experimental import pallas as pl
from jax.experimental.pallas import tpu as pltpu
from jax.experimental.pallas import tpu_sc as plsc

N = 8192
K = 8192
D = 32
TN = 256
TKC = 512
NT = N // TN
KC = K // TKC


def _dist_body(f_ref, et_ref, minv_ref, mini_ref):
    f = f_ref[...]

    def step(j, carry):
        mv, mi = carry
        et = et_ref[:, pl.ds(j * TKC, TKC)]
        e2 = jnp.sum(et * et, axis=0)
        prod = lax.dot_general(f, et, (((1,), (0,)), ((), ())),
                               preferred_element_type=jnp.float32)
        s = e2[None, :] - 2.0 * prod
        tmin = jnp.min(s, axis=1, keepdims=True)
        cols = lax.broadcasted_iota(jnp.int32, s.shape, 1)
        targ = jnp.min(jnp.where(s == tmin, cols, K), axis=1) + j * TKC
        tmin = tmin[:, 0]
        better = tmin < mv
        return jnp.where(better, tmin, mv), jnp.where(better, targ, mi)

    mv0 = jnp.full((TN,), jnp.inf, dtype=jnp.float32)
    mi0 = jnp.zeros((TN,), dtype=jnp.int32)
    mv, mi = lax.fori_loop(0, KC, step, (mv0, mi0))
    x2 = jnp.sum(f * f, axis=1)
    minv_ref[0, 0, :] = jnp.maximum(mv + x2, 0.0)
    mini_ref[0, 0, :] = mi


def _dist_argmin(f, et):
    minv, mini = pl.pallas_call(
        _dist_body,
        grid=(NT,),
        in_specs=[
            pl.BlockSpec((TN, D), lambda n: (n, 0)),
            pl.BlockSpec((D, K), lambda n: (0, 0)),
        ],
        out_specs=[
            pl.BlockSpec((1, 1, TN), lambda n: (n, 0, 0)),
            pl.BlockSpec((1, 1, TN), lambda n: (n, 0, 0)),
        ],
        out_shape=[
            jax.ShapeDtypeStruct((NT, 1, TN), jnp.float32),
            jax.ShapeDtypeStruct((NT, 1, TN), jnp.int32),
        ],
    )(f, et)
    return minv.reshape(N), mini.reshape(N)


_SC_INFO = plsc.get_sparse_core_info()
_NW = _SC_INFO.num_cores * _SC_INFO.num_subcores
_BPW = N // _NW


@functools.partial(
    pl.kernel,
    mesh=plsc.VectorSubcoreMesh(core_axis_name="c", subcore_axis_name="s"),
    out_type=jax.ShapeDtypeStruct((N, D), jnp.float32),
    scratch_types=[
        pltpu.VMEM((_BPW,), jnp.int32),
        pltpu.VMEM((_BPW, D), jnp.float32),
        pltpu.SemaphoreType.DMA,
    ],
    compiler_params=pltpu.CompilerParams(use_tc_tiling_on_sc=False),
)
def _sc_gather(table_hbm, idx_hbm, out_hbm, idx_v, rows_v, sem):
    wid = lax.axis_index("s") * _SC_INFO.num_cores + lax.axis_index("c")
    base = wid * _BPW
    pltpu.sync_copy(idx_hbm.at[pl.ds(base, _BPW)], idx_v)
    pltpu.async_copy(table_hbm.at[idx_v], rows_v, sem).wait()
    pltpu.sync_copy(rows_v, out_hbm.at[pl.ds(base, _BPW)])


def kernel(x, embed):
    x = x.astype(jnp.float32)
    f = x.reshape(N, D)
    e = embed.reshape(K, D).astype(jnp.float32)
    minv, mini = _dist_argmin(f, e.T)
    quantize = _sc_gather(e, mini)
    return quantize, mini.reshape(1, N), minv.reshape(1, N)


# speedup vs baseline: 1.2770x; 1.2770x over previous
import functools

import jax
import jax.numpy as jnp
from jax import lax
from jax.experimental import pallas as pl
from jax.experimental.pallas import tpu as pltpu
from jax.experimental.pallas import tpu_sc as plsc

N = 8192
K = 8192
D = 32
TN = 256
TKC = 512
NT = N // TN
KC = K // TKC


DA = 40


def _dist_body(f_ref, et_ref, minv_ref, mini_ref, aug_ref):
    n = pl.program_id(0)

    @pl.when(n == 0)
    def _():
        et = et_ref[...]
        aug_ref[0:D, :] = -2.0 * et
        aug_ref[D, :] = jnp.sum(et * et, axis=0)
        aug_ref[D + 1:DA, :] = jnp.zeros((DA - D - 1, K), jnp.float32)

    f = f_ref[...]
    fa = jnp.concatenate(
        [f, jnp.ones((TN, 1), jnp.float32), jnp.zeros((TN, DA - D - 1), jnp.float32)],
        axis=1)

    def step(j, carry):
        mv, mc = carry
        aug = aug_ref[:, pl.ds(j * TKC, TKC)]
        s = lax.dot_general(fa, aug, (((1,), (0,)), ((), ())),
                            preferred_element_type=jnp.float32)
        better = s < mv
        return jnp.where(better, s, mv), jnp.where(better, j, mc)

    mv0 = jnp.full((TN, TKC), jnp.inf, dtype=jnp.float32)
    mc0 = jnp.zeros((TN, TKC), dtype=jnp.int32)
    mv, mc = lax.fori_loop(0, KC, step, (mv0, mc0))

    idx = mc * TKC + lax.broadcasted_iota(jnp.int32, (TN, TKC), 1)
    m = jnp.min(mv, axis=1, keepdims=True)
    targ = jnp.min(jnp.where(mv == m, idx, K), axis=1)
    x2 = jnp.sum(f * f, axis=1)
    minv_ref[0, 0, :] = jnp.maximum(m[:, 0] + x2, 0.0)
    mini_ref[0, 0, :] = targ


def _dist_argmin(f, et):
    minv, mini = pl.pallas_call(
        _dist_body,
        grid=(NT,),
        in_specs=[
            pl.BlockSpec((TN, D), lambda n: (n, 0)),
            pl.BlockSpec((D, K), lambda n: (0, 0)),
        ],
        out_specs=[
            pl.BlockSpec((1, 1, TN), lambda n: (n, 0, 0)),
            pl.BlockSpec((1, 1, TN), lambda n: (n, 0, 0)),
        ],
        out_shape=[
            jax.ShapeDtypeStruct((NT, 1, TN), jnp.float32),
            jax.ShapeDtypeStruct((NT, 1, TN), jnp.int32),
        ],
        scratch_shapes=[pltpu.VMEM((DA, K), jnp.float32)],
    )(f, et)
    return minv.reshape(N), mini.reshape(N)


_SC_INFO = plsc.get_sparse_core_info()
_NW = _SC_INFO.num_cores * _SC_INFO.num_subcores
_BPW = N // _NW


@functools.partial(
    pl.kernel,
    mesh=plsc.VectorSubcoreMesh(core_axis_name="c", subcore_axis_name="s"),
    out_type=jax.ShapeDtypeStruct((N, D), jnp.float32),
    scratch_types=[
        pltpu.VMEM((_BPW,), jnp.int32),
        pltpu.VMEM((_BPW, D), jnp.float32),
        pltpu.SemaphoreType.DMA,
    ],
    compiler_params=pltpu.CompilerParams(use_tc_tiling_on_sc=False),
)
def _sc_gather(table_hbm, idx_hbm, out_hbm, idx_v, rows_v, sem):
    wid = lax.axis_index("s") * _SC_INFO.num_cores + lax.axis_index("c")
    base = wid * _BPW
    pltpu.sync_copy(idx_hbm.at[pl.ds(base, _BPW)], idx_v)
    pltpu.async_copy(table_hbm.at[idx_v], rows_v, sem).wait()
    pltpu.sync_copy(rows_v, out_hbm.at[pl.ds(base, _BPW)])


def kernel(x, embed):
    x = x.astype(jnp.float32)
    f = x.reshape(N, D)
    e = embed.reshape(K, D).astype(jnp.float32)
    minv, mini = _dist_argmin(f, e.T)
    quantize = _sc_gather(e, mini)
    return quantize, mini.reshape(1, N), minv.reshape(1, N)
